# Initial kernel scaffold; baseline (speedup 1.0000x reference)
import jax
import jax.numpy as jnp
from jax.experimental import pallas as pl


def kernel(feature_maps_0, feature_maps_1, feature_maps_2, feature_maps_3, rois):
    raise NotImplementedError("write your pallas kernel here")


# speedup vs baseline: 38.6824x; 38.6824x over previous
import dataclasses
import functools

import jax
import jax.numpy as jnp
import numpy as np
from jax import lax
from jax.experimental import pallas as pl
from jax.experimental.pallas import tpu as pltpu
from jax.experimental.pallas import tpu_sc as plsc

OUT = 7
S = 16
NPAD = 1024
RPW = 32
NW = 32
SIZES = (256, 128, 64, 32)
STRIDES = (4, 8, 16, 32)

_IDIV_M = [0] + [-(-256 // k) for k in range(1, 8)]


def _meta_kernel(rois_ref, mi_ref, mf_ref, gtl_ref, gtr_ref, gbl_ref,
                 gbr_ref, oix_ref):
    rois = rois_ref[...]
    n = rois.shape[0]
    x1 = rois[:, 1:2]
    y1 = rois[:, 2:3]
    x2 = rois[:, 3:4]
    y2 = rois[:, 4:5]
    roi_h = y2 - y1
    roi_w = x2 - x1
    lvl_f = jnp.log(jnp.sqrt(roi_h * roi_w) / 224.0) / jnp.log(2.0)
    level = jnp.clip(jnp.round(4.0 + lvl_f).astype(jnp.int32), 2, 5)
    box_b = rois[:, 0:1].astype(jnp.int32)

    i7 = lax.broadcasted_iota(jnp.int32, (1, OUT), 1).astype(jnp.float32)

    def per_level(li):
        stride = STRIDES[li]
        h = SIZES[li]
        by1 = x1 * (1.0 / stride)
        bx1 = y1 * (1.0 / stride)
        by2 = x2 * (1.0 / stride)
        bx2 = y2 * (1.0 / stride)
        hs = (by2 - by1) * (h - 1) / (OUT - 1)
        ws = (bx2 - bx1) * (h - 1) / (OUT - 1)
        in_y = by1 * (h - 1) + i7 * hs
        in_x = bx1 * (h - 1) + i7 * ws
        vy = (in_y >= 0) & (in_y <= h - 1)
        vx = (in_x >= 0) & (in_x <= h - 1)
        ny = vy.astype(jnp.int32).sum(axis=1, keepdims=True)
        nx = vx.astype(jnp.int32).sum(axis=1, keepdims=True)
        return ny, nx, by1, bx1, hs, ws

    vals = [per_level(li) for li in range(4)]

    def sel(idx, dtype=None):
        out = vals[0][idx]
        for li in range(1, 4):
            out = jnp.where(level == li + 2, vals[li][idx], out)
        return out

    ny = sel(0)
    nx = sel(1)
    y1c = sel(2)
    x1c = sel(3)
    hs = sel(4)
    ws = sel(5)
    hm1f = jnp.where(level == 2, 255.0,
           jnp.where(level == 3, 127.0,
           jnp.where(level == 4, 63.0, 31.0)))
    hm1i = hm1f.astype(jnp.int32)
    hi = hm1i + 1
    cnt = ny * nx

    t16i = lax.broadcasted_iota(jnp.int32, (1, S), 1)
    nxm = jnp.maximum(nx, 1)
    i16 = jnp.zeros((n, S), jnp.int32)
    for k in range(1, 8):
        i16 = jnp.where(nxm == k, (t16i * _IDIV_M[k]) >> 8, i16)
    j16 = t16i - i16 * nxm
    valid = t16i < cnt

    i16f = i16.astype(jnp.float32)
    j16f = j16.astype(jnp.float32)
    in_y = y1c * hm1f + i16f * hs
    in_x = x1c * hm1f + j16f * ws
    top = jnp.floor(in_y)
    bot = jnp.ceil(in_y)
    lef = jnp.floor(in_x)
    rig = jnp.ceil(in_x)
    yl = in_y - top
    xl = in_x - lef
    ti = jnp.clip(top, 0, hm1f).astype(jnp.int32)
    bi = jnp.clip(bot, 0, hm1f).astype(jnp.int32)
    li = jnp.clip(lef, 0, hm1f).astype(jnp.int32)
    ri = jnp.clip(rig, 0, hm1f).astype(jnp.int32)
    rowt = (box_b * hi + ti) * hi
    rowb = (box_b * hi + bi) * hi
    idx_tl = rowt + li
    idx_tr = rowt + ri
    idx_bl = rowb + li
    idx_br = rowb + ri
    vf = valid.astype(jnp.float32)
    wtl = (1.0 - xl) * (1.0 - yl) * vf
    wtr = xl * (1.0 - yl) * vf
    wbl = (1.0 - xl) * yl * vf
    wbr = xl * yl * vf
    rid = lax.broadcasted_iota(jnp.int32, (n, S), 0)
    oidx = jnp.where(valid, rid * 49 + i16 * 7 + j16, rid * 49 + 48)

    gtl_ref[0:n, :] = idx_tl
    gtr_ref[0:n, :] = idx_tr
    gbl_ref[0:n, :] = idx_bl
    gbr_ref[0:n, :] = idx_br
    oix_ref[0:n, :] = oidx
    for ref in (gtl_ref, gtr_ref, gbl_ref, gbr_ref, oix_ref):
        ref[n:NPAD, :] = jnp.zeros((NPAD - n, 16), jnp.int32)
    mi_ref[0:n, 0:16] = idx_tl
    mi_ref[0:n, 16:32] = idx_tr
    mi_ref[0:n, 32:48] = idx_bl
    mi_ref[0:n, 48:64] = idx_br
    mi_ref[0:n, 64:80] = oidx
    mi_ref[0:n, 80:96] = jnp.broadcast_to(cnt, (n, 16))
    mi_ref[0:n, 96:112] = jnp.broadcast_to(level, (n, 16))
    mi_ref[0:n, 112:128] = jnp.zeros((n, 16), jnp.int32)
    mi_ref[n:NPAD, :] = jnp.zeros((NPAD - n, 128), jnp.int32)
    mf_ref[0:n, 0:16] = wtl
    mf_ref[0:n, 16:32] = wtr
    mf_ref[0:n, 32:48] = wbl
    mf_ref[0:n, 48:64] = wbr
    mf_ref[0:n, 64:128] = jnp.zeros((n, 64), jnp.float32)
    mf_ref[n:NPAD, :] = jnp.zeros((NPAD - n, 128), jnp.float32)


def _sc_kernel(mi_hbm, mf_hbm, gtl_hbm, gtr_hbm, gbl_hbm, gbr_hbm, oix_hbm,
               f0, f1, f2, f3, out_hbm,
               mi_v, mf_v, gib0, gib1, gib2, gib3, oib,
               tl_v, tr_v, bl_v, br_v, orow_v, zero_v,
               semz, semm, semg, semo):
    nrows_out = out_hbm.shape[0]
    wid = lax.axis_index("s") * 2 + lax.axis_index("c")
    base = wid * RPW
    frefs = (f0, f1, f2, f3)
    gibs = (gib0, gib1, gib2, gib3)
    cbufs = (tl_v, tr_v, bl_v, br_v)
    lane = lax.iota(jnp.int32, 16)
    zrow = jnp.zeros((16,), jnp.float32)

    @pl.loop(0, zero_v.shape[0])
    def _(i):
        @pl.loop(0, 16)
        def _(ch):
            zero_v[i, pl.ds(ch * 16, 16)] = zrow

    nzdma = RPW * 49 // 56
    for k in range(nzdma):
        @pl.when(base * 49 + k * 56 < nrows_out)
        def _(k=k):
            pltpu.async_copy(
                zero_v, out_hbm.at[pl.ds(base * 49 + k * 56, 56)], semz)

    mcps = [pltpu.async_copy(src.at[pl.ds(base, RPW)], dst, semm)
            for src, dst in ((mi_hbm, mi_v), (mf_hbm, mf_v),
                             (gtl_hbm, gib0), (gtr_hbm, gib1),
                             (gbl_hbm, gib2), (gbr_hbm, gib3),
                             (oix_hbm, oib))]
    for cp in mcps:
        cp.wait()

    for k in range(nzdma):
        @pl.when(base * 49 + k * 56 < nrows_out)
        def _(k=k):
            pltpu.make_async_copy(
                zero_v, out_hbm.at[pl.ds(base * 49 + k * 56, 56)],
                semz).wait()

    @pl.loop(0, RPW)
    def _(r):
        cnt = jnp.max(mi_v[r, pl.ds(80, 16)])

        @pl.when(cnt > 0)
        def _():
            lvl = jnp.max(mi_v[r, pl.ds(96, 16)])
            for L in range(4):
                @pl.when(lvl == L + 2)
                def _(L=L):
                    cps = [pltpu.async_copy(frefs[L].at[gibs[c].at[r]],
                                            cbufs[c], semg)
                           for c in range(4)]
                    for cp in cps:
                        cp.wait()
            wr = [mf_v[r, pl.ds(c * 16, 16)] for c in range(4)]

            @pl.loop(0, S)
            def _(s):
                spl = [jnp.ones((16,), jnp.float32)
                       * jnp.max(jnp.where(lane == s, wr[c], -1.0))
                       for c in range(4)]

                @pl.loop(0, 16)
                def _(ch):
                    off = ch * 16
                    val = (tl_v[s, pl.ds(off, 16)] * spl[0]
                           + tr_v[s, pl.ds(off, 16)] * spl[1]
                           + bl_v[s, pl.ds(off, 16)] * spl[2]
                           + br_v[s, pl.ds(off, 16)] * spl[3])
                    orow_v[s, pl.ds(off, 16)] = val

            pltpu.async_copy(orow_v, out_hbm.at[oib.at[r]], semo).wait()


def kernel(feature_maps_0, feature_maps_1, feature_maps_2, feature_maps_3,
           rois):
    n = rois.shape[0]
    c = feature_maps_0.shape[-1]
    feats2d = [f.reshape(-1, c) for f in
               (feature_maps_0, feature_maps_1, feature_maps_2,
                feature_maps_3)]

    mi = pl.pallas_call(
        _meta_kernel,
        out_shape=[
            jax.ShapeDtypeStruct((NPAD, 128), jnp.int32),
            jax.ShapeDtypeStruct((NPAD, 128), jnp.float32),
        ] + [jax.ShapeDtypeStruct((NPAD, 16), jnp.int32)] * 5,
    )(rois)

    cp = pltpu.CompilerParams()
    if "needs_layout_passes" in pltpu.CompilerParams.__dataclass_fields__:
        cp = dataclasses.replace(cp, needs_layout_passes=False)
    mesh = plsc.VectorSubcoreMesh(core_axis_name="c", subcore_axis_name="s")
    sc = pl.kernel(
        _sc_kernel,
        out_type=jax.ShapeDtypeStruct((n * 49, c), jnp.float32),
        mesh=mesh,
        compiler_params=cp,
        scratch_types=[
            pltpu.VMEM((RPW, 128), jnp.int32),
            pltpu.VMEM((RPW, 128), jnp.float32),
            pltpu.VMEM((RPW, 16), jnp.int32),
            pltpu.VMEM((RPW, 16), jnp.int32),
            pltpu.VMEM((RPW, 16), jnp.int32),
            pltpu.VMEM((RPW, 16), jnp.int32),
            pltpu.VMEM((RPW, 16), jnp.int32),
            pltpu.VMEM((S, 256), jnp.float32),
            pltpu.VMEM((S, 256), jnp.float32),
            pltpu.VMEM((S, 256), jnp.float32),
            pltpu.VMEM((S, 256), jnp.float32),
            pltpu.VMEM((S, 256), jnp.float32),
            pltpu.VMEM((56, 256), jnp.float32),
            pltpu.SemaphoreType.DMA,
            pltpu.SemaphoreType.DMA,
            pltpu.SemaphoreType.DMA,
            pltpu.SemaphoreType.DMA,
        ],
    )
    out2d = sc(*mi, *feats2d)
    return out2d.reshape(n, OUT, OUT, c)


# speedup vs baseline: 67.9047x; 1.7554x over previous
import dataclasses
import functools

import jax
import jax.numpy as jnp
import numpy as np
from jax import lax
from jax.experimental import pallas as pl
from jax.experimental.pallas import tpu as pltpu
from jax.experimental.pallas import tpu_sc as plsc

OUT = 7
S = 16
NPAD = 1024
RPW = 32
NW = 32
SIZES = (256, 128, 64, 32)
STRIDES = (4, 8, 16, 32)

_IDIV_M = [0] + [-(-256 // k) for k in range(1, 8)]


def _meta_kernel(rois_ref, mi_ref, mf_ref, gtl_ref, gtr_ref, gbl_ref,
                 gbr_ref, oix_ref):
    rois = rois_ref[...]
    n = rois.shape[0]
    x1 = rois[:, 1:2]
    y1 = rois[:, 2:3]
    x2 = rois[:, 3:4]
    y2 = rois[:, 4:5]
    roi_h = y2 - y1
    roi_w = x2 - x1
    lvl_f = jnp.log(jnp.sqrt(roi_h * roi_w) / 224.0) / jnp.log(2.0)
    level = jnp.clip(jnp.round(4.0 + lvl_f).astype(jnp.int32), 2, 5)
    box_b = rois[:, 0:1].astype(jnp.int32)

    i7 = lax.broadcasted_iota(jnp.int32, (1, OUT), 1).astype(jnp.float32)

    def per_level(li):
        stride = STRIDES[li]
        h = SIZES[li]
        by1 = x1 * (1.0 / stride)
        bx1 = y1 * (1.0 / stride)
        by2 = x2 * (1.0 / stride)
        bx2 = y2 * (1.0 / stride)
        hs = (by2 - by1) * (h - 1) / (OUT - 1)
        ws = (bx2 - bx1) * (h - 1) / (OUT - 1)
        in_y = by1 * (h - 1) + i7 * hs
        in_x = bx1 * (h - 1) + i7 * ws
        vy = (in_y >= 0) & (in_y <= h - 1)
        vx = (in_x >= 0) & (in_x <= h - 1)
        ny = vy.astype(jnp.int32).sum(axis=1, keepdims=True)
        nx = vx.astype(jnp.int32).sum(axis=1, keepdims=True)
        return ny, nx, by1, bx1, hs, ws

    vals = [per_level(li) for li in range(4)]

    def sel(idx, dtype=None):
        out = vals[0][idx]
        for li in range(1, 4):
            out = jnp.where(level == li + 2, vals[li][idx], out)
        return out

    ny = sel(0)
    nx = sel(1)
    y1c = sel(2)
    x1c = sel(3)
    hs = sel(4)
    ws = sel(5)
    hm1f = jnp.where(level == 2, 255.0,
           jnp.where(level == 3, 127.0,
           jnp.where(level == 4, 63.0, 31.0)))
    hm1i = hm1f.astype(jnp.int32)
    hi = hm1i + 1
    cnt = ny * nx

    t16i = lax.broadcasted_iota(jnp.int32, (1, S), 1)
    nxm = jnp.maximum(nx, 1)
    i16 = jnp.zeros((n, S), jnp.int32)
    for k in range(1, 8):
        i16 = jnp.where(nxm == k, (t16i * _IDIV_M[k]) >> 8, i16)
    j16 = t16i - i16 * nxm
    valid = t16i < cnt

    i16f = i16.astype(jnp.float32)
    j16f = j16.astype(jnp.float32)
    in_y = y1c * hm1f + i16f * hs
    in_x = x1c * hm1f + j16f * ws
    top = jnp.floor(in_y)
    bot = jnp.ceil(in_y)
    lef = jnp.floor(in_x)
    rig = jnp.ceil(in_x)
    yl = in_y - top
    xl = in_x - lef
    ti = jnp.clip(top, 0, hm1f).astype(jnp.int32)
    bi = jnp.clip(bot, 0, hm1f).astype(jnp.int32)
    li = jnp.clip(lef, 0, hm1f).astype(jnp.int32)
    ri = jnp.clip(rig, 0, hm1f).astype(jnp.int32)
    rowt = (box_b * hi + ti) * hi
    rowb = (box_b * hi + bi) * hi
    idx_tl = rowt + li
    idx_tr = rowt + ri
    idx_bl = rowb + li
    idx_br = rowb + ri
    vf = valid.astype(jnp.float32)
    wtl = (1.0 - xl) * (1.0 - yl) * vf
    wtr = xl * (1.0 - yl) * vf
    wbl = (1.0 - xl) * yl * vf
    wbr = xl * yl * vf
    rid = lax.broadcasted_iota(jnp.int32, (n, S), 0)
    oidx = jnp.where(valid, rid * 49 + i16 * 7 + j16, rid * 49 + 48)

    gtl_ref[0:n, :] = idx_tl
    gtr_ref[0:n, :] = idx_tr
    gbl_ref[0:n, :] = idx_bl
    gbr_ref[0:n, :] = idx_br
    oix_ref[0:n, :] = oidx
    for ref in (gtl_ref, gtr_ref, gbl_ref, gbr_ref, oix_ref):
        ref[n:NPAD, :] = jnp.zeros((NPAD - n, 16), jnp.int32)
    mi_ref[0:n, 0:16] = idx_tl
    mi_ref[0:n, 16:32] = idx_tr
    mi_ref[0:n, 32:48] = idx_bl
    mi_ref[0:n, 48:64] = idx_br
    mi_ref[0:n, 64:80] = oidx
    mi_ref[0:n, 80:96] = jnp.broadcast_to(cnt, (n, 16))
    mi_ref[0:n, 96:112] = jnp.broadcast_to(level, (n, 16))
    mi_ref[0:n, 112:120] = jnp.broadcast_to(ny, (n, 8))
    mi_ref[0:n, 120:128] = jnp.broadcast_to(nx, (n, 8))
    mi_ref[n:NPAD, :] = jnp.zeros((NPAD - n, 128), jnp.int32)
    mf_ref[0:n, 0:16] = wtl
    mf_ref[0:n, 16:32] = wtr
    mf_ref[0:n, 32:48] = wbl
    mf_ref[0:n, 48:64] = wbr
    mf_ref[0:n, 64:128] = jnp.zeros((n, 64), jnp.float32)
    mf_ref[n:NPAD, :] = jnp.zeros((NPAD - n, 128), jnp.float32)


def _sc_kernel(mi_hbm, mf_hbm, gtl_hbm, gtr_hbm, gbl_hbm, gbr_hbm, oix_hbm,
               f0, f1, f2, f3, out_hbm,
               mi_v, mf_v, gib0, gib1, gib2, gib3, oib,
               tl_v, tr_v, bl_v, br_v, orow_v,
               semm, semg, semo):
    wid = lax.axis_index("s") * 2 + lax.axis_index("c")
    base = wid * RPW
    frefs = (f0, f1, f2, f3)
    gibs = (gib0, gib1, gib2, gib3)
    cbufs = (tl_v, tr_v, bl_v, br_v)
    lane = lax.iota(jnp.int32, 16)

    mcps = [pltpu.async_copy(src.at[pl.ds(base, RPW)], dst, semm)
            for src, dst in ((mi_hbm, mi_v), (mf_hbm, mf_v),
                             (gtl_hbm, gib0), (gtr_hbm, gib1),
                             (gbl_hbm, gib2), (gbr_hbm, gib3),
                             (oix_hbm, oib))]
    for cp in mcps:
        cp.wait()

    @pl.loop(0, RPW)
    def _(r):
        cnt = jnp.max(mi_v[r, pl.ds(80, 16)])

        @pl.when(cnt > 0)
        def _():
            lvl = jnp.max(mi_v[r, pl.ds(96, 16)])
            for L in range(4):
                @pl.when(lvl == L + 2)
                def _(L=L):
                    cps = [pltpu.async_copy(frefs[L].at[gibs[c].at[r]],
                                            cbufs[c], semg)
                           for c in range(4)]
                    for cp in cps:
                        cp.wait()
            wr = [mf_v[r, pl.ds(c * 16, 16)] for c in range(4)]

            @pl.loop(0, S)
            def _(s):
                spl = [jnp.ones((16,), jnp.float32)
                       * jnp.max(jnp.where(lane == s, wr[c], -1.0))
                       for c in range(4)]

                @pl.loop(0, 16)
                def _(ch):
                    off = ch * 16
                    val = (tl_v[s, pl.ds(off, 16)] * spl[0]
                           + tr_v[s, pl.ds(off, 16)] * spl[1]
                           + bl_v[s, pl.ds(off, 16)] * spl[2]
                           + br_v[s, pl.ds(off, 16)] * spl[3])
                    orow_v[s, pl.ds(off, 16)] = val

            pltpu.async_copy(orow_v, out_hbm.at[oib.at[r]], semo).wait()


_FIN_R = 8


def _finalize_kernel(out2d_ref, mi_ref, out_ref):
    j7 = lax.broadcasted_iota(jnp.int32, (OUT, 1), 0)
    for r in range(_FIN_R):
        ny = mi_ref[r:r + 1, 112:113]
        nx = mi_ref[r:r + 1, 120:121]
        mj = j7 < nx
        for i in range(OUT):
            rows = out2d_ref[pl.ds(r * 49 + i * 7, OUT), :]
            mask = mj & (i < ny)
            out_ref[r, i, :, :] = jnp.where(mask, rows, 0.0)


def kernel(feature_maps_0, feature_maps_1, feature_maps_2, feature_maps_3,
           rois):
    n = rois.shape[0]
    c = feature_maps_0.shape[-1]
    feats2d = [f.reshape(-1, c) for f in
               (feature_maps_0, feature_maps_1, feature_maps_2,
                feature_maps_3)]

    mi = pl.pallas_call(
        _meta_kernel,
        out_shape=[
            jax.ShapeDtypeStruct((NPAD, 128), jnp.int32),
            jax.ShapeDtypeStruct((NPAD, 128), jnp.float32),
        ] + [jax.ShapeDtypeStruct((NPAD, 16), jnp.int32)] * 5,
    )(rois)

    cp = pltpu.CompilerParams()
    if "needs_layout_passes" in pltpu.CompilerParams.__dataclass_fields__:
        cp = dataclasses.replace(cp, needs_layout_passes=False)
    mesh = plsc.VectorSubcoreMesh(core_axis_name="c", subcore_axis_name="s")
    sc = pl.kernel(
        _sc_kernel,
        out_type=jax.ShapeDtypeStruct((n * 49, c), jnp.float32),
        mesh=mesh,
        compiler_params=cp,
        scratch_types=[
            pltpu.VMEM((RPW, 128), jnp.int32),
            pltpu.VMEM((RPW, 128), jnp.float32),
            pltpu.VMEM((RPW, 16), jnp.int32),
            pltpu.VMEM((RPW, 16), jnp.int32),
            pltpu.VMEM((RPW, 16), jnp.int32),
            pltpu.VMEM((RPW, 16), jnp.int32),
            pltpu.VMEM((RPW, 16), jnp.int32),
            pltpu.VMEM((S, 256), jnp.float32),
            pltpu.VMEM((S, 256), jnp.float32),
            pltpu.VMEM((S, 256), jnp.float32),
            pltpu.VMEM((S, 256), jnp.float32),
            pltpu.VMEM((S, 256), jnp.float32),
            pltpu.SemaphoreType.DMA,
            pltpu.SemaphoreType.DMA,
            pltpu.SemaphoreType.DMA,
        ],
    )
    out2d = sc(*mi, *feats2d)

    nblk = n // _FIN_R
    return pl.pallas_call(
        _finalize_kernel,
        grid=(nblk,),
        in_specs=[
            pl.BlockSpec((_FIN_R * 49, c), lambda i: (i, 0)),
            pl.BlockSpec((_FIN_R, 128), lambda i: (i, 0)),
        ],
        out_specs=pl.BlockSpec((_FIN_R, OUT, OUT, c),
                               lambda i: (i, 0, 0, 0)),
        out_shape=jax.ShapeDtypeStruct((n, OUT, OUT, c), jnp.float32),
    )(out2d, mi[0])


# speedup vs baseline: 77.3054x; 1.1384x over previous
import dataclasses
import functools

import jax
import jax.numpy as jnp
import numpy as np
from jax import lax
from jax.experimental import pallas as pl
from jax.experimental.pallas import tpu as pltpu
from jax.experimental.pallas import tpu_sc as plsc

OUT = 7
S = 16
NPAD = 1024
RPW = 32
NW = 32
SIZES = (256, 128, 64, 32)
STRIDES = (4, 8, 16, 32)

_IDIV_M = [0] + [-(-256 // k) for k in range(1, 8)]


def _meta_kernel(rois_ref, mi_ref, mf_ref, gtl_ref, gtr_ref, gbl_ref,
                 gbr_ref, oix_ref):
    rois = rois_ref[...]
    n = rois.shape[0]
    x1 = rois[:, 1:2]
    y1 = rois[:, 2:3]
    x2 = rois[:, 3:4]
    y2 = rois[:, 4:5]
    roi_h = y2 - y1
    roi_w = x2 - x1
    lvl_f = jnp.log(jnp.sqrt(roi_h * roi_w) / 224.0) / jnp.log(2.0)
    level = jnp.clip(jnp.round(4.0 + lvl_f).astype(jnp.int32), 2, 5)
    box_b = rois[:, 0:1].astype(jnp.int32)

    i7 = lax.broadcasted_iota(jnp.int32, (1, OUT), 1).astype(jnp.float32)

    def per_level(li):
        stride = STRIDES[li]
        h = SIZES[li]
        by1 = x1 * (1.0 / stride)
        bx1 = y1 * (1.0 / stride)
        by2 = x2 * (1.0 / stride)
        bx2 = y2 * (1.0 / stride)
        hs = (by2 - by1) * (h - 1) / (OUT - 1)
        ws = (bx2 - bx1) * (h - 1) / (OUT - 1)
        in_y = by1 * (h - 1) + i7 * hs
        in_x = bx1 * (h - 1) + i7 * ws
        vy = (in_y >= 0) & (in_y <= h - 1)
        vx = (in_x >= 0) & (in_x <= h - 1)
        ny = vy.astype(jnp.int32).sum(axis=1, keepdims=True)
        nx = vx.astype(jnp.int32).sum(axis=1, keepdims=True)
        return ny, nx, by1, bx1, hs, ws

    vals = [per_level(li) for li in range(4)]

    def sel(idx, dtype=None):
        out = vals[0][idx]
        for li in range(1, 4):
            out = jnp.where(level == li + 2, vals[li][idx], out)
        return out

    ny = sel(0)
    nx = sel(1)
    y1c = sel(2)
    x1c = sel(3)
    hs = sel(4)
    ws = sel(5)
    hm1f = jnp.where(level == 2, 255.0,
           jnp.where(level == 3, 127.0,
           jnp.where(level == 4, 63.0, 31.0)))
    hm1i = hm1f.astype(jnp.int32)
    hi = hm1i + 1
    cnt = ny * nx

    t16i = lax.broadcasted_iota(jnp.int32, (1, S), 1)
    nxm = jnp.maximum(nx, 1)
    i16 = jnp.zeros((n, S), jnp.int32)
    for k in range(1, 8):
        i16 = jnp.where(nxm == k, (t16i * _IDIV_M[k]) >> 8, i16)
    j16 = t16i - i16 * nxm
    valid = t16i < cnt

    i16f = i16.astype(jnp.float32)
    j16f = j16.astype(jnp.float32)
    in_y = y1c * hm1f + i16f * hs
    in_x = x1c * hm1f + j16f * ws
    top = jnp.floor(in_y)
    bot = jnp.ceil(in_y)
    lef = jnp.floor(in_x)
    rig = jnp.ceil(in_x)
    yl = in_y - top
    xl = in_x - lef
    ti = jnp.clip(top, 0, hm1f).astype(jnp.int32)
    bi = jnp.clip(bot, 0, hm1f).astype(jnp.int32)
    li = jnp.clip(lef, 0, hm1f).astype(jnp.int32)
    ri = jnp.clip(rig, 0, hm1f).astype(jnp.int32)
    rowt = (box_b * hi + ti) * hi
    rowb = (box_b * hi + bi) * hi
    idx_tl = rowt + li
    idx_tr = rowt + ri
    idx_bl = rowb + li
    idx_br = rowb + ri
    vf = valid.astype(jnp.float32)
    wtl = (1.0 - xl) * (1.0 - yl) * vf
    wtr = xl * (1.0 - yl) * vf
    wbl = (1.0 - xl) * yl * vf
    wbr = xl * yl * vf
    rid = lax.broadcasted_iota(jnp.int32, (n, S), 0)
    oidx = jnp.where(valid, rid * 49 + i16 * 7 + j16, rid * 49 + 48)
    n49 = 49 * n

    gtl_ref[0:n, :] = idx_tl
    gtr_ref[0:n, :] = idx_tr
    gbl_ref[0:n, :] = idx_bl
    gbr_ref[0:n, :] = idx_br
    oix_ref[0:n, 0:16] = oidx
    oix_ref[0:n, 16:32] = oidx + n49
    oix_ref[n:NPAD, :] = jnp.zeros((NPAD - n, 32), jnp.int32)
    for ref in (gtl_ref, gtr_ref, gbl_ref, gbr_ref):
        ref[n:NPAD, :] = jnp.zeros((NPAD - n, 16), jnp.int32)
    mi_ref[0:n, 0:16] = idx_tl
    mi_ref[0:n, 16:32] = idx_tr
    mi_ref[0:n, 32:48] = idx_bl
    mi_ref[0:n, 48:64] = idx_br
    mi_ref[0:n, 64:80] = oidx
    mi_ref[0:n, 80:96] = jnp.broadcast_to(cnt, (n, 16))
    mi_ref[0:n, 96:112] = jnp.broadcast_to(level, (n, 16))
    mi_ref[0:n, 112:120] = jnp.broadcast_to(ny, (n, 8))
    mi_ref[0:n, 120:128] = jnp.broadcast_to(nx, (n, 8))
    mi_ref[n:NPAD, :] = jnp.zeros((NPAD - n, 128), jnp.int32)
    mf_ref[0:n, 0:16] = wtl
    mf_ref[0:n, 16:32] = wtr
    mf_ref[0:n, 32:48] = wbl
    mf_ref[0:n, 48:64] = wbr
    mf_ref[0:n, 64:128] = jnp.zeros((n, 64), jnp.float32)
    mf_ref[n:NPAD, :] = jnp.zeros((NPAD - n, 128), jnp.float32)


def _sc_kernel(mi_hbm, mf_hbm, gtl_hbm, gtr_hbm, gbl_hbm, gbr_hbm, oix_hbm,
               f0, f1, f2, f3, out_hbm,
               mi_v, mf_v, gib0, gib1, gib2, gib3, oib,
               tl_v, tr_v, bl_v, br_v, orow_v,
               semm, semg, semo):
    wid = lax.axis_index("s") * 2 + lax.axis_index("c")
    base = wid * RPW
    frefs = (f0, f1, f2, f3)
    gibs = (gib0, gib1, gib2, gib3)
    cbufs = (tl_v, tr_v, bl_v, br_v)
    lane = lax.iota(jnp.int32, 16)

    mcps = [pltpu.async_copy(src.at[pl.ds(base, RPW)], dst, semm)
            for src, dst in ((mi_hbm, mi_v), (mf_hbm, mf_v),
                             (gtl_hbm, gib0), (gtr_hbm, gib1),
                             (gbl_hbm, gib2), (gbr_hbm, gib3),
                             (oix_hbm, oib))]
    for cp in mcps:
        cp.wait()

    @pl.loop(0, RPW)
    def _(r):
        cnt = jnp.max(mi_v[r, pl.ds(80, 16)])

        @pl.when(cnt > 0)
        def _():
            lvl = jnp.max(mi_v[r, pl.ds(96, 16)])
            for L in range(4):
                @pl.when(lvl == L + 2)
                def _(L=L):
                    cps = [pltpu.async_copy(frefs[L].at[gibs[c].at[r]],
                                            cbufs[c], semg)
                           for c in range(4)]
                    for cp in cps:
                        cp.wait()
            wr = [mf_v[r, pl.ds(c * 16, 16)] for c in range(4)]

            @pl.loop(0, S)
            def _(s):
                spl = [jnp.ones((16,), jnp.float32)
                       * jnp.max(jnp.where(lane == s, wr[c], -1.0))
                       for c in range(4)]

                @pl.loop(0, 16)
                def _(ch):
                    off = ch * 16
                    val = (tl_v[s, pl.ds(off, 16)] * spl[0]
                           + tr_v[s, pl.ds(off, 16)] * spl[1]
                           + bl_v[s, pl.ds(off, 16)] * spl[2]
                           + br_v[s, pl.ds(off, 16)] * spl[3])
                    orow_v[s + S * (ch >> 3), pl.ds((ch & 7) * 16, 16)] = val

            pltpu.async_copy(orow_v, out_hbm.at[oib.at[r]], semo).wait()


_FIN_R = 40


def _finalize_kernel(lo_ref, hi_ref, mi_ref, out_ref):
    p49 = lax.broadcasted_iota(jnp.int32, (49, 1), 0)
    i49 = (p49 * 37) >> 8
    j49 = p49 - i49 * 7
    for r in range(_FIN_R):
        ny = mi_ref[r:r + 1, 112:113]
        nx = mi_ref[r:r + 1, 120:121]
        mask = (i49 < ny) & (j49 < nx)
        lo = jnp.where(mask, lo_ref[pl.ds(r * 49, 49), :], 0.0)
        hi = jnp.where(mask, hi_ref[pl.ds(r * 49, 49), :], 0.0)
        for i in range(OUT):
            out_ref[r, i, :, 0:128] = lo[i * 7:i * 7 + 7, :]
            out_ref[r, i, :, 128:256] = hi[i * 7:i * 7 + 7, :]


def kernel(feature_maps_0, feature_maps_1, feature_maps_2, feature_maps_3,
           rois):
    n = rois.shape[0]
    c = feature_maps_0.shape[-1]
    feats2d = [f.reshape(-1, c) for f in
               (feature_maps_0, feature_maps_1, feature_maps_2,
                feature_maps_3)]

    mi = pl.pallas_call(
        _meta_kernel,
        out_shape=[
            jax.ShapeDtypeStruct((NPAD, 128), jnp.int32),
            jax.ShapeDtypeStruct((NPAD, 128), jnp.float32),
        ] + [jax.ShapeDtypeStruct((NPAD, 16), jnp.int32)] * 4
          + [jax.ShapeDtypeStruct((NPAD, 32), jnp.int32)],
    )(rois)

    cp = pltpu.CompilerParams()
    if "needs_layout_passes" in pltpu.CompilerParams.__dataclass_fields__:
        cp = dataclasses.replace(cp, needs_layout_passes=False)
    mesh = plsc.VectorSubcoreMesh(core_axis_name="c", subcore_axis_name="s")
    sc = pl.kernel(
        _sc_kernel,
        out_type=jax.ShapeDtypeStruct((2 * n * 49, c // 2), jnp.float32),
        mesh=mesh,
        compiler_params=cp,
        scratch_types=[
            pltpu.VMEM((RPW, 128), jnp.int32),
            pltpu.VMEM((RPW, 128), jnp.float32),
            pltpu.VMEM((RPW, 16), jnp.int32),
            pltpu.VMEM((RPW, 16), jnp.int32),
            pltpu.VMEM((RPW, 16), jnp.int32),
            pltpu.VMEM((RPW, 16), jnp.int32),
            pltpu.VMEM((RPW, 32), jnp.int32),
            pltpu.VMEM((S, 256), jnp.float32),
            pltpu.VMEM((S, 256), jnp.float32),
            pltpu.VMEM((S, 256), jnp.float32),
            pltpu.VMEM((S, 256), jnp.float32),
            pltpu.VMEM((2 * S, 128), jnp.float32),
            pltpu.SemaphoreType.DMA,
            pltpu.SemaphoreType.DMA,
            pltpu.SemaphoreType.DMA,
        ],
    )
    out2d = sc(*mi, *feats2d)

    nblk = n // _FIN_R
    return pl.pallas_call(
        _finalize_kernel,
        grid=(nblk,),
        in_specs=[
            pl.BlockSpec((_FIN_R * 49, c // 2), lambda i: (i, 0)),
            pl.BlockSpec((_FIN_R * 49, c // 2),
                         lambda i: (i + nblk, 0)),
            pl.BlockSpec((_FIN_R, 128), lambda i: (i, 0)),
        ],
        out_specs=pl.BlockSpec((_FIN_R, OUT, OUT, c),
                               lambda i: (i, 0, 0, 0)),
        out_shape=jax.ShapeDtypeStruct((n, OUT, OUT, c), jnp.float32),
    )(out2d, out2d, mi[0])


# speedup vs baseline: 111.9333x; 1.4479x over previous
import dataclasses
import functools

import jax
import jax.numpy as jnp
import numpy as np
from jax import lax
from jax.experimental import pallas as pl
from jax.experimental.pallas import tpu as pltpu
from jax.experimental.pallas import tpu_sc as plsc

OUT = 7
S = 16
NPAD = 1024
RPW = 32
NW = 32
SIZES = (256, 128, 64, 32)
STRIDES = (4, 8, 16, 32)

_IDIV_M = [0] + [-(-256 // k) for k in range(1, 8)]


def _meta_kernel(rois_ref, mi_ref, mf_ref, gtl_ref, gtr_ref, gbl_ref,
                 gbr_ref, oix_ref):
    rois = rois_ref[...]
    n = rois.shape[0]
    x1 = rois[:, 1:2]
    y1 = rois[:, 2:3]
    x2 = rois[:, 3:4]
    y2 = rois[:, 4:5]
    roi_h = y2 - y1
    roi_w = x2 - x1
    lvl_f = jnp.log(jnp.sqrt(roi_h * roi_w) / 224.0) / jnp.log(2.0)
    level = jnp.clip(jnp.round(4.0 + lvl_f).astype(jnp.int32), 2, 5)
    box_b = rois[:, 0:1].astype(jnp.int32)

    i7 = lax.broadcasted_iota(jnp.int32, (1, OUT), 1).astype(jnp.float32)

    def per_level(li):
        stride = STRIDES[li]
        h = SIZES[li]
        by1 = x1 * (1.0 / stride)
        bx1 = y1 * (1.0 / stride)
        by2 = x2 * (1.0 / stride)
        bx2 = y2 * (1.0 / stride)
        hs = (by2 - by1) * (h - 1) / (OUT - 1)
        ws = (bx2 - bx1) * (h - 1) / (OUT - 1)
        in_y = by1 * (h - 1) + i7 * hs
        in_x = bx1 * (h - 1) + i7 * ws
        vy = (in_y >= 0) & (in_y <= h - 1)
        vx = (in_x >= 0) & (in_x <= h - 1)
        ny = vy.astype(jnp.int32).sum(axis=1, keepdims=True)
        nx = vx.astype(jnp.int32).sum(axis=1, keepdims=True)
        return ny, nx, by1, bx1, hs, ws

    vals = [per_level(li) for li in range(4)]

    def sel(idx, dtype=None):
        out = vals[0][idx]
        for li in range(1, 4):
            out = jnp.where(level == li + 2, vals[li][idx], out)
        return out

    ny = sel(0)
    nx = sel(1)
    y1c = sel(2)
    x1c = sel(3)
    hs = sel(4)
    ws = sel(5)
    hm1f = jnp.where(level == 2, 255.0,
           jnp.where(level == 3, 127.0,
           jnp.where(level == 4, 63.0, 31.0)))
    hm1i = hm1f.astype(jnp.int32)
    hi = hm1i + 1
    cnt = ny * nx

    t16i = lax.broadcasted_iota(jnp.int32, (1, S), 1)
    nxm = jnp.maximum(nx, 1)
    i16 = jnp.zeros((n, S), jnp.int32)
    for k in range(1, 8):
        i16 = jnp.where(nxm == k, (t16i * _IDIV_M[k]) >> 8, i16)
    j16 = t16i - i16 * nxm
    valid = t16i < cnt

    i16f = i16.astype(jnp.float32)
    j16f = j16.astype(jnp.float32)
    in_y = y1c * hm1f + i16f * hs
    in_x = x1c * hm1f + j16f * ws
    top = jnp.floor(in_y)
    bot = jnp.ceil(in_y)
    lef = jnp.floor(in_x)
    rig = jnp.ceil(in_x)
    yl = in_y - top
    xl = in_x - lef
    ti = jnp.clip(top, 0, hm1f).astype(jnp.int32)
    bi = jnp.clip(bot, 0, hm1f).astype(jnp.int32)
    li = jnp.clip(lef, 0, hm1f).astype(jnp.int32)
    ri = jnp.clip(rig, 0, hm1f).astype(jnp.int32)
    rowt = (box_b * hi + ti) * hi
    rowb = (box_b * hi + bi) * hi
    idx_tl = rowt + li
    idx_tr = rowt + ri
    idx_bl = rowb + li
    idx_br = rowb + ri
    vf = valid.astype(jnp.float32)
    wtl = (1.0 - xl) * (1.0 - yl) * vf
    wtr = xl * (1.0 - yl) * vf
    wbl = (1.0 - xl) * yl * vf
    wbr = xl * yl * vf
    rid = lax.broadcasted_iota(jnp.int32, (n, S), 0)
    oidx = jnp.where(valid, (i16 * 7 + j16) * n + rid, 48 * n + rid)
    n49 = 49 * n

    gtl_ref[0:n, :] = idx_tl
    gtr_ref[0:n, :] = idx_tr
    gbl_ref[0:n, :] = idx_bl
    gbr_ref[0:n, :] = idx_br
    oix_ref[0:n, 0:16] = oidx
    oix_ref[0:n, 16:32] = oidx + n49
    oix_ref[n:NPAD, :] = jnp.zeros((NPAD - n, 32), jnp.int32)
    for ref in (gtl_ref, gtr_ref, gbl_ref, gbr_ref):
        ref[n:NPAD, :] = jnp.zeros((NPAD - n, 16), jnp.int32)
    mi_ref[0:n, 0:16] = idx_tl
    mi_ref[0:n, 16:32] = idx_tr
    mi_ref[0:n, 32:48] = idx_bl
    mi_ref[0:n, 48:64] = idx_br
    mi_ref[0:n, 64:80] = oidx
    mi_ref[0:n, 80:96] = jnp.broadcast_to(cnt, (n, 16))
    mi_ref[0:n, 96:112] = jnp.broadcast_to(level, (n, 16))
    mi_ref[0:n, 112:120] = jnp.broadcast_to(ny, (n, 8))
    mi_ref[0:n, 120:128] = jnp.broadcast_to(nx, (n, 8))
    mi_ref[n:NPAD, :] = jnp.zeros((NPAD - n, 128), jnp.int32)
    mf_ref[0:n, 0:16] = wtl
    mf_ref[0:n, 16:32] = wtr
    mf_ref[0:n, 32:48] = wbl
    mf_ref[0:n, 48:64] = wbr
    mf_ref[0:n, 64:128] = jnp.zeros((n, 64), jnp.float32)
    mf_ref[n:NPAD, :] = jnp.zeros((NPAD - n, 128), jnp.float32)


def _sc_kernel(mi_hbm, mf_hbm, gtl_hbm, gtr_hbm, gbl_hbm, gbr_hbm, oix_hbm,
               f0, f1, f2, f3, out_hbm,
               mi_v, mf_v, gib0, gib1, gib2, gib3, oib,
               tl_v, tr_v, bl_v, br_v, orow_v,
               semm, semg, semo):
    wid = lax.axis_index("s") * 2 + lax.axis_index("c")
    base = wid * RPW
    frefs = (f0, f1, f2, f3)
    gibs = (gib0, gib1, gib2, gib3)
    cbufs = (tl_v, tr_v, bl_v, br_v)
    lane = lax.iota(jnp.int32, 16)

    mcps = [pltpu.async_copy(src.at[pl.ds(base, RPW)], dst, semm)
            for src, dst in ((mi_hbm, mi_v), (mf_hbm, mf_v),
                             (gtl_hbm, gib0), (gtr_hbm, gib1),
                             (gbl_hbm, gib2), (gbr_hbm, gib3),
                             (oix_hbm, oib))]
    for cp in mcps:
        cp.wait()

    @pl.loop(0, RPW)
    def _(r):
        cnt = jnp.max(mi_v[r, pl.ds(80, 16)])

        @pl.when(cnt > 0)
        def _():
            lvl = jnp.max(mi_v[r, pl.ds(96, 16)])
            for L in range(4):
                @pl.when(lvl == L + 2)
                def _(L=L):
                    cps = [pltpu.async_copy(frefs[L].at[gibs[c].at[r]],
                                            cbufs[c], semg)
                           for c in range(4)]
                    for cp in cps:
                        cp.wait()
            wr = [mf_v[r, pl.ds(c * 16, 16)] for c in range(4)]

            @pl.loop(0, S)
            def _(s):
                spl = [jnp.ones((16,), jnp.float32)
                       * jnp.max(jnp.where(lane == s, wr[c], -1.0))
                       for c in range(4)]

                @pl.loop(0, 16)
                def _(ch):
                    off = ch * 16
                    val = (tl_v[s, pl.ds(off, 16)] * spl[0]
                           + tr_v[s, pl.ds(off, 16)] * spl[1]
                           + bl_v[s, pl.ds(off, 16)] * spl[2]
                           + br_v[s, pl.ds(off, 16)] * spl[3])
                    orow_v[s + S * (ch >> 3), pl.ds((ch & 7) * 16, 16)] = val

            pltpu.async_copy(orow_v, out_hbm.at[oib.at[r]], semo).wait()


def _finalize_kernel(lo_ref, hi_ref, mi_ref, out_ref):
    n = lo_ref.shape[0]
    p = pl.program_id(0)
    i_p = (p * 37) >> 8
    j_p = p - i_p * 7
    ny = mi_ref[0:n, 112:113]
    nx = mi_ref[0:n, 120:121]
    mask = (i_p < ny) & (j_p < nx)
    out_ref[0, 0, :, 0:128] = jnp.where(mask, lo_ref[...], 0.0)
    out_ref[0, 0, :, 128:256] = jnp.where(mask, hi_ref[...], 0.0)


def kernel(feature_maps_0, feature_maps_1, feature_maps_2, feature_maps_3,
           rois):
    n = rois.shape[0]
    c = feature_maps_0.shape[-1]
    feats2d = [f.reshape(-1, c) for f in
               (feature_maps_0, feature_maps_1, feature_maps_2,
                feature_maps_3)]

    mi = pl.pallas_call(
        _meta_kernel,
        out_shape=[
            jax.ShapeDtypeStruct((NPAD, 128), jnp.int32),
            jax.ShapeDtypeStruct((NPAD, 128), jnp.float32),
        ] + [jax.ShapeDtypeStruct((NPAD, 16), jnp.int32)] * 4
          + [jax.ShapeDtypeStruct((NPAD, 32), jnp.int32)],
    )(rois)

    cp = pltpu.CompilerParams()
    if "needs_layout_passes" in pltpu.CompilerParams.__dataclass_fields__:
        cp = dataclasses.replace(cp, needs_layout_passes=False)
    mesh = plsc.VectorSubcoreMesh(core_axis_name="c", subcore_axis_name="s")
    sc = pl.kernel(
        _sc_kernel,
        out_type=jax.ShapeDtypeStruct((2 * n * 49, c // 2), jnp.float32),
        mesh=mesh,
        compiler_params=cp,
        scratch_types=[
            pltpu.VMEM((RPW, 128), jnp.int32),
            pltpu.VMEM((RPW, 128), jnp.float32),
            pltpu.VMEM((RPW, 16), jnp.int32),
            pltpu.VMEM((RPW, 16), jnp.int32),
            pltpu.VMEM((RPW, 16), jnp.int32),
            pltpu.VMEM((RPW, 16), jnp.int32),
            pltpu.VMEM((RPW, 32), jnp.int32),
            pltpu.VMEM((S, 256), jnp.float32),
            pltpu.VMEM((S, 256), jnp.float32),
            pltpu.VMEM((S, 256), jnp.float32),
            pltpu.VMEM((S, 256), jnp.float32),
            pltpu.VMEM((2 * S, 128), jnp.float32),
            pltpu.SemaphoreType.DMA,
            pltpu.SemaphoreType.DMA,
            pltpu.SemaphoreType.DMA,
        ],
    )
    out2d = sc(*mi, *feats2d)

    outp = pl.pallas_call(
        _finalize_kernel,
        grid=(49,),
        in_specs=[
            pl.BlockSpec((n, c // 2), lambda p: (p, 0)),
            pl.BlockSpec((n, c // 2), lambda p: (p + 49, 0)),
            pl.BlockSpec((NPAD, 128), lambda p: (0, 0)),
        ],
        out_specs=pl.BlockSpec((1, 1, n, c), lambda p: (p // 7, p % 7, 0, 0)),
        out_shape=jax.ShapeDtypeStruct((OUT, OUT, n, c), jnp.float32),
    )(out2d, out2d, mi[0])
    return jnp.transpose(outp, (2, 0, 1, 3))


# speedup vs baseline: 138.1509x; 1.2342x over previous
import dataclasses
import functools

import jax
import jax.numpy as jnp
import numpy as np
from jax import lax
from jax.experimental import pallas as pl
from jax.experimental.pallas import tpu as pltpu
from jax.experimental.pallas import tpu_sc as plsc

OUT = 7
S = 16
NPAD = 1024
RPW = 32
NW = 32
SIZES = (256, 128, 64, 32)
STRIDES = (4, 8, 16, 32)

_IDIV_M = [0] + [-(-256 // k) for k in range(1, 8)]


def _meta_kernel(rois_ref, mi_ref, mf_ref, gtl_ref, gtr_ref, gbl_ref,
                 gbr_ref, oix_ref):
    rois = rois_ref[...]
    n = rois.shape[0]
    x1 = rois[:, 1:2]
    y1 = rois[:, 2:3]
    x2 = rois[:, 3:4]
    y2 = rois[:, 4:5]
    roi_h = y2 - y1
    roi_w = x2 - x1
    lvl_f = jnp.log(jnp.sqrt(roi_h * roi_w) / 224.0) / jnp.log(2.0)
    level = jnp.clip(jnp.round(4.0 + lvl_f).astype(jnp.int32), 2, 5)
    box_b = rois[:, 0:1].astype(jnp.int32)

    i7 = lax.broadcasted_iota(jnp.int32, (1, OUT), 1).astype(jnp.float32)

    def per_level(li):
        stride = STRIDES[li]
        h = SIZES[li]
        by1 = x1 * (1.0 / stride)
        bx1 = y1 * (1.0 / stride)
        by2 = x2 * (1.0 / stride)
        bx2 = y2 * (1.0 / stride)
        hs = (by2 - by1) * (h - 1) / (OUT - 1)
        ws = (bx2 - bx1) * (h - 1) / (OUT - 1)
        in_y = by1 * (h - 1) + i7 * hs
        in_x = bx1 * (h - 1) + i7 * ws
        vy = (in_y >= 0) & (in_y <= h - 1)
        vx = (in_x >= 0) & (in_x <= h - 1)
        ny = vy.astype(jnp.int32).sum(axis=1, keepdims=True)
        nx = vx.astype(jnp.int32).sum(axis=1, keepdims=True)
        return ny, nx, by1, bx1, hs, ws

    vals = [per_level(li) for li in range(4)]

    def sel(idx, dtype=None):
        out = vals[0][idx]
        for li in range(1, 4):
            out = jnp.where(level == li + 2, vals[li][idx], out)
        return out

    ny = sel(0)
    nx = sel(1)
    y1c = sel(2)
    x1c = sel(3)
    hs = sel(4)
    ws = sel(5)
    hm1f = jnp.where(level == 2, 255.0,
           jnp.where(level == 3, 127.0,
           jnp.where(level == 4, 63.0, 31.0)))
    hm1i = hm1f.astype(jnp.int32)
    hi = hm1i + 1
    cnt = ny * nx

    t16i = lax.broadcasted_iota(jnp.int32, (1, S), 1)
    nxm = jnp.maximum(nx, 1)
    i16 = jnp.zeros((n, S), jnp.int32)
    for k in range(1, 8):
        i16 = jnp.where(nxm == k, (t16i * _IDIV_M[k]) >> 8, i16)
    j16 = t16i - i16 * nxm
    valid = t16i < cnt

    i16f = i16.astype(jnp.float32)
    j16f = j16.astype(jnp.float32)
    in_y = y1c * hm1f + i16f * hs
    in_x = x1c * hm1f + j16f * ws
    top = jnp.floor(in_y)
    bot = jnp.ceil(in_y)
    lef = jnp.floor(in_x)
    rig = jnp.ceil(in_x)
    yl = in_y - top
    xl = in_x - lef
    ti = jnp.clip(top, 0, hm1f).astype(jnp.int32)
    bi = jnp.clip(bot, 0, hm1f).astype(jnp.int32)
    li = jnp.clip(lef, 0, hm1f).astype(jnp.int32)
    ri = jnp.clip(rig, 0, hm1f).astype(jnp.int32)
    rowt = (box_b * hi + ti) * hi
    rowb = (box_b * hi + bi) * hi
    idx_tl = rowt + li
    idx_tr = rowt + ri
    idx_bl = rowb + li
    idx_br = rowb + ri
    vf = valid.astype(jnp.float32)
    wtl = (1.0 - xl) * (1.0 - yl) * vf
    wtr = xl * (1.0 - yl) * vf
    wbl = (1.0 - xl) * yl * vf
    wbr = xl * yl * vf
    rid = lax.broadcasted_iota(jnp.int32, (n, S), 0)
    oidx = jnp.where(valid, (i16 * 7 + j16) * n + rid, 48 * n + rid)
    n49 = 49 * n

    gtl_ref[0:n, :] = idx_tl
    gtr_ref[0:n, :] = idx_tr
    gbl_ref[0:n, :] = idx_bl
    gbr_ref[0:n, :] = idx_br
    oix_ref[0:n, 0:16] = oidx
    oix_ref[0:n, 16:32] = oidx + n49
    oix_ref[n:NPAD, :] = jnp.zeros((NPAD - n, 32), jnp.int32)
    for ref in (gtl_ref, gtr_ref, gbl_ref, gbr_ref):
        ref[n:NPAD, :] = jnp.zeros((NPAD - n, 16), jnp.int32)
    mi_ref[0:n, 0:16] = idx_tl
    mi_ref[0:n, 16:32] = idx_tr
    mi_ref[0:n, 32:48] = idx_bl
    mi_ref[0:n, 48:64] = idx_br
    mi_ref[0:n, 64:80] = oidx
    mi_ref[0:n, 80:96] = jnp.broadcast_to(cnt, (n, 16))
    mi_ref[0:n, 96:112] = jnp.broadcast_to(level, (n, 16))
    mi_ref[0:n, 112:120] = jnp.broadcast_to(ny, (n, 8))
    mi_ref[0:n, 120:128] = jnp.broadcast_to(nx, (n, 8))
    mi_ref[n:NPAD, :] = jnp.zeros((NPAD - n, 128), jnp.int32)
    mf_ref[0:n, 0:16] = wtl
    mf_ref[0:n, 16:32] = wtr
    mf_ref[0:n, 32:48] = wbl
    mf_ref[0:n, 48:64] = wbr
    mf_ref[0:n, 64:128] = jnp.zeros((n, 64), jnp.float32)
    mf_ref[n:NPAD, :] = jnp.zeros((NPAD - n, 128), jnp.float32)


def _sc_kernel(mi_hbm, mf_hbm, gtl_hbm, gtr_hbm, gbl_hbm, gbr_hbm, oix_hbm,
               f0, f1, f2, f3, out_hbm,
               mi_v, mf_v, gib0, gib1, gib2, gib3, oib,
               tl_v, tr_v, bl_v, br_v, orow_v,
               semm, semg, semo):
    wid = lax.axis_index("s") * 2 + lax.axis_index("c")
    base = wid * RPW
    frefs = (f0, f1, f2, f3)
    gibs = (gib0, gib1, gib2, gib3)
    cbufs = (tl_v, tr_v, bl_v, br_v)
    lane = lax.iota(jnp.int32, 16)

    mcps = [pltpu.async_copy(src.at[pl.ds(base, RPW)], dst, semm)
            for src, dst in ((mi_hbm, mi_v), (mf_hbm, mf_v),
                             (gtl_hbm, gib0), (gtr_hbm, gib1),
                             (gbl_hbm, gib2), (gbr_hbm, gib3),
                             (oix_hbm, oib))]
    for cp in mcps:
        cp.wait()

    @pl.loop(0, RPW)
    def _(r):
        cnt = jnp.max(mi_v[r, pl.ds(80, 16)])

        @pl.when(cnt > 0)
        def _():
            lvl = jnp.max(mi_v[r, pl.ds(96, 16)])
            for L in range(4):
                @pl.when(lvl == L + 2)
                def _(L=L):
                    cps = [pltpu.async_copy(frefs[L].at[gibs[c].at[r]],
                                            cbufs[c], semg)
                           for c in range(4)]
                    for cp in cps:
                        cp.wait()
            wr = [mf_v[r, pl.ds(c * 16, 16)] for c in range(4)]

            @pl.loop(0, S)
            def _(s):
                spl = [jnp.ones((16,), jnp.float32)
                       * jnp.max(jnp.where(lane == s, wr[c], -1.0))
                       for c in range(4)]

                @pl.loop(0, 16)
                def _(ch):
                    off = ch * 16
                    val = (tl_v[s, pl.ds(off, 16)] * spl[0]
                           + tr_v[s, pl.ds(off, 16)] * spl[1]
                           + bl_v[s, pl.ds(off, 16)] * spl[2]
                           + br_v[s, pl.ds(off, 16)] * spl[3])
                    orow_v[s + S * (ch >> 3), pl.ds((ch & 7) * 16, 16)] = val

            pltpu.async_copy(orow_v, out_hbm.at[oib.at[r]], semo).wait()


def _finalize_kernel(lo_ref, hi_ref, mi_ref, out_ref):
    n = lo_ref.shape[0] // OUT
    i_p = pl.program_id(0)
    ny = mi_ref[0:n, 112:113]
    nx = mi_ref[0:n, 120:128]
    mi_row = i_p < ny
    for j in range(OUT):
        mask = mi_row & (j < nx[:, j:j + 1])
        out_ref[0, j, :, 0:128] = jnp.where(
            mask, lo_ref[pl.ds(j * n, n), :], 0.0)
        out_ref[0, j, :, 128:256] = jnp.where(
            mask, hi_ref[pl.ds(j * n, n), :], 0.0)


def kernel(feature_maps_0, feature_maps_1, feature_maps_2, feature_maps_3,
           rois):
    n = rois.shape[0]
    c = feature_maps_0.shape[-1]
    feats2d = [f.reshape(-1, c) for f in
               (feature_maps_0, feature_maps_1, feature_maps_2,
                feature_maps_3)]

    mi = pl.pallas_call(
        _meta_kernel,
        out_shape=[
            jax.ShapeDtypeStruct((NPAD, 128), jnp.int32),
            jax.ShapeDtypeStruct((NPAD, 128), jnp.float32),
        ] + [jax.ShapeDtypeStruct((NPAD, 16), jnp.int32)] * 4
          + [jax.ShapeDtypeStruct((NPAD, 32), jnp.int32)],
    )(rois)

    cp = pltpu.CompilerParams()
    if "needs_layout_passes" in pltpu.CompilerParams.__dataclass_fields__:
        cp = dataclasses.replace(cp, needs_layout_passes=False)
    mesh = plsc.VectorSubcoreMesh(core_axis_name="c", subcore_axis_name="s")
    sc = pl.kernel(
        _sc_kernel,
        out_type=jax.ShapeDtypeStruct((2 * n * 49, c // 2), jnp.float32),
        mesh=mesh,
        compiler_params=cp,
        scratch_types=[
            pltpu.VMEM((RPW, 128), jnp.int32),
            pltpu.VMEM((RPW, 128), jnp.float32),
            pltpu.VMEM((RPW, 16), jnp.int32),
            pltpu.VMEM((RPW, 16), jnp.int32),
            pltpu.VMEM((RPW, 16), jnp.int32),
            pltpu.VMEM((RPW, 16), jnp.int32),
            pltpu.VMEM((RPW, 32), jnp.int32),
            pltpu.VMEM((S, 256), jnp.float32),
            pltpu.VMEM((S, 256), jnp.float32),
            pltpu.VMEM((S, 256), jnp.float32),
            pltpu.VMEM((S, 256), jnp.float32),
            pltpu.VMEM((2 * S, 128), jnp.float32),
            pltpu.SemaphoreType.DMA,
            pltpu.SemaphoreType.DMA,
            pltpu.SemaphoreType.DMA,
        ],
    )
    out2d = sc(*mi, *feats2d)

    outp = pl.pallas_call(
        _finalize_kernel,
        grid=(OUT,),
        in_specs=[
            pl.BlockSpec((OUT * n, c // 2), lambda p: (p, 0)),
            pl.BlockSpec((OUT * n, c // 2), lambda p: (p + OUT, 0)),
            pl.BlockSpec((NPAD, 128), lambda p: (0, 0)),
        ],
        out_specs=pl.BlockSpec((1, OUT, n, c), lambda p: (p, 0, 0, 0)),
        out_shape=jax.ShapeDtypeStruct((OUT, OUT, n, c), jnp.float32),
    )(out2d, out2d, mi[0])
    return jnp.transpose(outp, (2, 0, 1, 3))


# speedup vs baseline: 162.0544x; 1.1730x over previous
import dataclasses
import functools

import jax
import jax.numpy as jnp
import numpy as np
from jax import lax
from jax.experimental import pallas as pl
from jax.experimental.pallas import tpu as pltpu
from jax.experimental.pallas import tpu_sc as plsc

OUT = 7
S = 16
NPAD = 1024
RPW = 32
NW = 32
SIZES = (256, 128, 64, 32)
STRIDES = (4, 8, 16, 32)

_IDIV_M = [0] + [-(-256 // k) for k in range(1, 8)]


def _meta_kernel(rois_ref, mi_ref, mf_ref, gtl_ref, gtr_ref, gbl_ref,
                 gbr_ref, oix_ref, pf_ref):
    rois = rois_ref[...]
    n = rois.shape[0]
    x1 = rois[:, 1:2]
    y1 = rois[:, 2:3]
    x2 = rois[:, 3:4]
    y2 = rois[:, 4:5]
    roi_h = y2 - y1
    roi_w = x2 - x1
    lvl_f = jnp.log(jnp.sqrt(roi_h * roi_w) / 224.0) / jnp.log(2.0)
    level = jnp.clip(jnp.round(4.0 + lvl_f).astype(jnp.int32), 2, 5)
    box_b = rois[:, 0:1].astype(jnp.int32)

    i7 = lax.broadcasted_iota(jnp.int32, (1, OUT), 1).astype(jnp.float32)

    def per_level(li):
        stride = STRIDES[li]
        h = SIZES[li]
        by1 = x1 * (1.0 / stride)
        bx1 = y1 * (1.0 / stride)
        by2 = x2 * (1.0 / stride)
        bx2 = y2 * (1.0 / stride)
        hs = (by2 - by1) * (h - 1) / (OUT - 1)
        ws = (bx2 - bx1) * (h - 1) / (OUT - 1)
        in_y = by1 * (h - 1) + i7 * hs
        in_x = bx1 * (h - 1) + i7 * ws
        vy = (in_y >= 0) & (in_y <= h - 1)
        vx = (in_x >= 0) & (in_x <= h - 1)
        ny = vy.astype(jnp.int32).sum(axis=1, keepdims=True)
        nx = vx.astype(jnp.int32).sum(axis=1, keepdims=True)
        return ny, nx, by1, bx1, hs, ws

    vals = [per_level(li) for li in range(4)]

    def sel(idx, dtype=None):
        out = vals[0][idx]
        for li in range(1, 4):
            out = jnp.where(level == li + 2, vals[li][idx], out)
        return out

    ny = sel(0)
    nx = sel(1)
    y1c = sel(2)
    x1c = sel(3)
    hs = sel(4)
    ws = sel(5)
    hm1f = jnp.where(level == 2, 255.0,
           jnp.where(level == 3, 127.0,
           jnp.where(level == 4, 63.0, 31.0)))
    hm1i = hm1f.astype(jnp.int32)
    hi = hm1i + 1
    cnt = ny * nx

    t16i = lax.broadcasted_iota(jnp.int32, (1, S), 1)
    nxm = jnp.maximum(nx, 1)
    i16 = jnp.zeros((n, S), jnp.int32)
    for k in range(1, 8):
        i16 = jnp.where(nxm == k, (t16i * _IDIV_M[k]) >> 8, i16)
    j16 = t16i - i16 * nxm
    valid = t16i < cnt

    i16f = i16.astype(jnp.float32)
    j16f = j16.astype(jnp.float32)
    in_y = y1c * hm1f + i16f * hs
    in_x = x1c * hm1f + j16f * ws
    top = jnp.floor(in_y)
    bot = jnp.ceil(in_y)
    lef = jnp.floor(in_x)
    rig = jnp.ceil(in_x)
    yl = in_y - top
    xl = in_x - lef
    ti = jnp.clip(top, 0, hm1f).astype(jnp.int32)
    bi = jnp.clip(bot, 0, hm1f).astype(jnp.int32)
    li = jnp.clip(lef, 0, hm1f).astype(jnp.int32)
    ri = jnp.clip(rig, 0, hm1f).astype(jnp.int32)
    rowt = (box_b * hi + ti) * hi
    rowb = (box_b * hi + bi) * hi
    idx_tl = rowt + li
    idx_tr = rowt + ri
    idx_bl = rowb + li
    idx_br = rowb + ri
    vf = valid.astype(jnp.float32)
    wtl = (1.0 - xl) * (1.0 - yl) * vf
    wtr = xl * (1.0 - yl) * vf
    wbl = (1.0 - xl) * yl * vf
    wbr = xl * yl * vf
    rid = lax.broadcasted_iota(jnp.int32, (n, S), 0)
    oidx = jnp.where(valid, (i16 * 7 + j16) * n + rid, 48 * n + rid)
    n49 = 49 * n

    gtl_ref[0:n, :] = idx_tl
    gtr_ref[0:n, :] = idx_tr
    gbl_ref[0:n, :] = idx_bl
    gbr_ref[0:n, :] = idx_br
    oix_ref[0:n, 0:16] = oidx
    oix_ref[0:n, 16:32] = oidx + n49
    oix_ref[n:NPAD, :] = jnp.zeros((NPAD - n, 32), jnp.int32)
    for ref in (gtl_ref, gtr_ref, gbl_ref, gbr_ref):
        ref[n:NPAD, :] = jnp.zeros((NPAD - n, 16), jnp.int32)
    mi_ref[0:n, 0:16] = idx_tl
    mi_ref[0:n, 16:32] = idx_tr
    mi_ref[0:n, 32:48] = idx_bl
    mi_ref[0:n, 48:64] = idx_br
    mi_ref[0:n, 64:80] = oidx
    mi_ref[0:n, 80:96] = jnp.broadcast_to(cnt, (n, 16))
    mi_ref[0:n, 96:112] = jnp.broadcast_to(level, (n, 16))
    mi_ref[0:n, 112:120] = jnp.broadcast_to(ny, (n, 8))
    mi_ref[0:n, 120:128] = jnp.broadcast_to(nx, (n, 8))
    mi_ref[n:NPAD, :] = jnp.zeros((NPAD - n, 128), jnp.int32)
    mf_ref[0:n, 0:16] = wtl
    mf_ref[0:n, 16:32] = wtr
    mf_ref[0:n, 32:48] = wbl
    mf_ref[0:n, 48:64] = wbr
    mf_ref[0:n, 64:128] = jnp.zeros((n, 64), jnp.float32)
    mf_ref[n:NPAD, :] = jnp.zeros((NPAD - n, 128), jnp.float32)

    p64 = lax.broadcasted_iota(jnp.int32, (n, 64), 1)
    i64 = (p64 * 37) >> 8
    j64 = p64 - i64 * 7
    pv = ((i64 < jnp.broadcast_to(ny, (n, 64)))
          & (j64 < jnp.broadcast_to(nx, (n, 64)))
          & (p64 < 49)).astype(jnp.int32)
    anyv = pv.max(axis=0, keepdims=True)
    pf_ref[...] = jnp.broadcast_to(anyv, (8, 64))


def _sc_kernel(mi_hbm, mf_hbm, gtl_hbm, gtr_hbm, gbl_hbm, gbr_hbm, oix_hbm,
               f0, f1, f2, f3, out_hbm,
               mi_v, mf_v, gib0, gib1, gib2, gib3, oib,
               tl_v, tr_v, bl_v, br_v, orow_v,
               semm, semg, semo):
    wid = lax.axis_index("s") * 2 + lax.axis_index("c")
    base = wid * RPW
    frefs = (f0, f1, f2, f3)
    gibs = (gib0, gib1, gib2, gib3)
    cbufs = (tl_v, tr_v, bl_v, br_v)
    lane = lax.iota(jnp.int32, 16)

    mcps = [pltpu.async_copy(src.at[pl.ds(base, RPW)], dst, semm)
            for src, dst in ((mi_hbm, mi_v), (mf_hbm, mf_v),
                             (gtl_hbm, gib0), (gtr_hbm, gib1),
                             (gbl_hbm, gib2), (gbr_hbm, gib3),
                             (oix_hbm, oib))]
    for cp in mcps:
        cp.wait()

    @pl.loop(0, RPW)
    def _(r):
        cnt = jnp.max(mi_v[r, pl.ds(80, 16)])

        @pl.when(cnt > 0)
        def _():
            lvl = jnp.max(mi_v[r, pl.ds(96, 16)])
            for L in range(4):
                @pl.when(lvl == L + 2)
                def _(L=L):
                    cps = [pltpu.async_copy(frefs[L].at[gibs[c].at[r]],
                                            cbufs[c], semg)
                           for c in range(4)]
                    for cp in cps:
                        cp.wait()
            wr = [mf_v[r, pl.ds(c * 16, 16)] for c in range(4)]

            @pl.loop(0, S)
            def _(s):
                spl = [jnp.ones((16,), jnp.float32)
                       * jnp.max(jnp.where(lane == s, wr[c], -1.0))
                       for c in range(4)]

                @pl.loop(0, 16)
                def _(ch):
                    off = ch * 16
                    val = (tl_v[s, pl.ds(off, 16)] * spl[0]
                           + tr_v[s, pl.ds(off, 16)] * spl[1]
                           + bl_v[s, pl.ds(off, 16)] * spl[2]
                           + br_v[s, pl.ds(off, 16)] * spl[3])
                    orow_v[s + S * (ch >> 3), pl.ds((ch & 7) * 16, 16)] = val

            pltpu.async_copy(orow_v, out_hbm.at[oib.at[r]], semo).wait()


def _finalize_kernel(pf_ref, o2d_hbm, mi_ref, out_ref, slo, shi, sem):
    n = out_ref.shape[2]
    i_p = pl.program_id(0)
    ny = mi_ref[0:n, 112:113]
    nx = mi_ref[0:n, 120:121]
    mi_row = i_p < ny
    zeros = jnp.zeros((n, 128), jnp.float32)
    for j in range(OUT):
        p = i_p * OUT + j

        @pl.when(pf_ref[p] > 0)
        def _(j=j, p=p):
            cl = pltpu.make_async_copy(
                o2d_hbm.at[pl.ds(p * n, n)], slo, sem)
            ch = pltpu.make_async_copy(
                o2d_hbm.at[pl.ds((49 + p) * n, n)], shi, sem)
            cl.start()
            ch.start()
            cl.wait()
            ch.wait()
            mask = mi_row & (j < nx)
            out_ref[0, j, :, 0:128] = jnp.where(mask, slo[...], 0.0)
            out_ref[0, j, :, 128:256] = jnp.where(mask, shi[...], 0.0)

        @pl.when(pf_ref[p] == 0)
        def _(j=j):
            out_ref[0, j, :, 0:128] = zeros
            out_ref[0, j, :, 128:256] = zeros


def kernel(feature_maps_0, feature_maps_1, feature_maps_2, feature_maps_3,
           rois):
    n = rois.shape[0]
    c = feature_maps_0.shape[-1]
    feats2d = [f.reshape(-1, c) for f in
               (feature_maps_0, feature_maps_1, feature_maps_2,
                feature_maps_3)]

    mi = pl.pallas_call(
        _meta_kernel,
        out_shape=[
            jax.ShapeDtypeStruct((NPAD, 128), jnp.int32),
            jax.ShapeDtypeStruct((NPAD, 128), jnp.float32),
        ] + [jax.ShapeDtypeStruct((NPAD, 16), jnp.int32)] * 4
          + [jax.ShapeDtypeStruct((NPAD, 32), jnp.int32),
             jax.ShapeDtypeStruct((8, 64), jnp.int32)],
    )(rois)

    cp = pltpu.CompilerParams()
    if "needs_layout_passes" in pltpu.CompilerParams.__dataclass_fields__:
        cp = dataclasses.replace(cp, needs_layout_passes=False)
    mesh = plsc.VectorSubcoreMesh(core_axis_name="c", subcore_axis_name="s")
    sc = pl.kernel(
        _sc_kernel,
        out_type=jax.ShapeDtypeStruct((2 * n * 49, c // 2), jnp.float32),
        mesh=mesh,
        compiler_params=cp,
        scratch_types=[
            pltpu.VMEM((RPW, 128), jnp.int32),
            pltpu.VMEM((RPW, 128), jnp.float32),
            pltpu.VMEM((RPW, 16), jnp.int32),
            pltpu.VMEM((RPW, 16), jnp.int32),
            pltpu.VMEM((RPW, 16), jnp.int32),
            pltpu.VMEM((RPW, 16), jnp.int32),
            pltpu.VMEM((RPW, 32), jnp.int32),
            pltpu.VMEM((S, 256), jnp.float32),
            pltpu.VMEM((S, 256), jnp.float32),
            pltpu.VMEM((S, 256), jnp.float32),
            pltpu.VMEM((S, 256), jnp.float32),
            pltpu.VMEM((2 * S, 128), jnp.float32),
            pltpu.SemaphoreType.DMA,
            pltpu.SemaphoreType.DMA,
            pltpu.SemaphoreType.DMA,
        ],
    )
    out2d = sc(*mi[:7], *feats2d)

    pflags = mi[7][0, :]
    outp = pl.pallas_call(
        _finalize_kernel,
        grid_spec=pltpu.PrefetchScalarGridSpec(
            num_scalar_prefetch=1,
            grid=(OUT,),
            in_specs=[
                pl.BlockSpec(memory_space=pl.ANY),
                pl.BlockSpec((NPAD, 128), lambda p, pf: (0, 0)),
            ],
            out_specs=pl.BlockSpec((1, OUT, n, c),
                                   lambda p, pf: (p, 0, 0, 0)),
            scratch_shapes=[
                pltpu.VMEM((n, 128), jnp.float32),
                pltpu.VMEM((n, 128), jnp.float32),
                pltpu.SemaphoreType.DMA,
            ],
        ),
        out_shape=jax.ShapeDtypeStruct((OUT, OUT, n, c), jnp.float32),
    )(pflags, out2d, mi[0])
    return jnp.transpose(outp, (2, 0, 1, 3))


# speedup vs baseline: 178.4545x; 1.1012x over previous
import dataclasses
import functools

import jax
import jax.numpy as jnp
import numpy as np
from jax import lax
from jax.experimental import pallas as pl
from jax.experimental.pallas import tpu as pltpu
from jax.experimental.pallas import tpu_sc as plsc

OUT = 7
S = 16
NPAD = 1024
RPW = 32
NW = 32
SIZES = (256, 128, 64, 32)
STRIDES = (4, 8, 16, 32)

_IDIV_M = [0] + [-(-256 // k) for k in range(1, 8)]


def _meta_kernel(rois_ref, mi_ref, mf_ref, gtl_ref, gtr_ref, gbl_ref,
                 gbr_ref, oix_ref, pf_ref):
    rois = rois_ref[...]
    n = rois.shape[0]
    x1 = rois[:, 1:2]
    y1 = rois[:, 2:3]
    x2 = rois[:, 3:4]
    y2 = rois[:, 4:5]
    roi_h = y2 - y1
    roi_w = x2 - x1
    lvl_f = jnp.log(jnp.sqrt(roi_h * roi_w) / 224.0) / jnp.log(2.0)
    level = jnp.clip(jnp.round(4.0 + lvl_f).astype(jnp.int32), 2, 5)
    box_b = rois[:, 0:1].astype(jnp.int32)

    i7 = lax.broadcasted_iota(jnp.int32, (1, OUT), 1).astype(jnp.float32)

    sinv = jnp.where(level == 2, 0.25,
           jnp.where(level == 3, 0.125,
           jnp.where(level == 4, 0.0625, 0.03125)))
    hm1f = jnp.where(level == 2, 255.0,
           jnp.where(level == 3, 127.0,
           jnp.where(level == 4, 63.0, 31.0)))
    hm1i = hm1f.astype(jnp.int32)
    hi = hm1i + 1
    y1c = x1 * sinv
    x1c = y1 * sinv
    y2c = x2 * sinv
    x2c = y2 * sinv
    hs = (y2c - y1c) * hm1f / (OUT - 1)
    ws = (x2c - x1c) * hm1f / (OUT - 1)
    in_y7 = y1c * hm1f + i7 * hs
    in_x7 = x1c * hm1f + i7 * ws
    vy = (in_y7 >= 0) & (in_y7 <= hm1f)
    vx = (in_x7 >= 0) & (in_x7 <= hm1f)
    ny = vy.astype(jnp.int32).sum(axis=1, keepdims=True)
    nx = vx.astype(jnp.int32).sum(axis=1, keepdims=True)
    cnt = ny * nx

    t16i = lax.broadcasted_iota(jnp.int32, (1, S), 1)
    nxm = jnp.maximum(nx, 1)
    i16 = jnp.zeros((n, S), jnp.int32)
    for k in range(1, 8):
        i16 = jnp.where(nxm == k, (t16i * _IDIV_M[k]) >> 8, i16)
    j16 = t16i - i16 * nxm
    valid = t16i < cnt

    i16f = i16.astype(jnp.float32)
    j16f = j16.astype(jnp.float32)
    in_y = y1c * hm1f + i16f * hs
    in_x = x1c * hm1f + j16f * ws
    top = jnp.floor(in_y)
    bot = jnp.ceil(in_y)
    lef = jnp.floor(in_x)
    rig = jnp.ceil(in_x)
    yl = in_y - top
    xl = in_x - lef
    ti = jnp.clip(top, 0, hm1f).astype(jnp.int32)
    bi = jnp.clip(bot, 0, hm1f).astype(jnp.int32)
    li = jnp.clip(lef, 0, hm1f).astype(jnp.int32)
    ri = jnp.clip(rig, 0, hm1f).astype(jnp.int32)
    rowt = (box_b * hi + ti) * hi
    rowb = (box_b * hi + bi) * hi
    idx_tl = rowt + li
    idx_tr = rowt + ri
    idx_bl = rowb + li
    idx_br = rowb + ri
    vf = valid.astype(jnp.float32)
    wtl = (1.0 - xl) * (1.0 - yl) * vf
    wtr = xl * (1.0 - yl) * vf
    wbl = (1.0 - xl) * yl * vf
    wbr = xl * yl * vf
    rid = lax.broadcasted_iota(jnp.int32, (n, S), 0)
    oidx = jnp.where(valid, (i16 * 7 + j16) * n + rid, 48 * n + rid)
    n49 = 49 * n

    gtl_ref[0:n, :] = idx_tl
    gtr_ref[0:n, :] = idx_tr
    gbl_ref[0:n, :] = idx_bl
    gbr_ref[0:n, :] = idx_br
    oix_ref[0:n, 0:16] = oidx
    oix_ref[0:n, 16:32] = oidx + n49
    oix_ref[n:NPAD, :] = jnp.zeros((NPAD - n, 32), jnp.int32)
    for ref in (gtl_ref, gtr_ref, gbl_ref, gbr_ref):
        ref[n:NPAD, :] = jnp.zeros((NPAD - n, 16), jnp.int32)
    mi_ref[0:n, 0:16] = idx_tl
    mi_ref[0:n, 16:32] = idx_tr
    mi_ref[0:n, 32:48] = idx_bl
    mi_ref[0:n, 48:64] = idx_br
    mi_ref[0:n, 64:80] = oidx
    mi_ref[0:n, 80:96] = jnp.broadcast_to(cnt, (n, 16))
    mi_ref[0:n, 96:112] = jnp.broadcast_to(level, (n, 16))
    mi_ref[0:n, 112:120] = jnp.broadcast_to(ny, (n, 8))
    mi_ref[0:n, 120:128] = jnp.broadcast_to(nx, (n, 8))
    mi_ref[n:NPAD, :] = jnp.zeros((NPAD - n, 128), jnp.int32)
    mf_ref[0:n, 0:16] = wtl
    mf_ref[0:n, 16:32] = wtr
    mf_ref[0:n, 32:48] = wbl
    mf_ref[0:n, 48:64] = wbr
    mf_ref[0:n, 64:128] = jnp.zeros((n, 64), jnp.float32)
    mf_ref[n:NPAD, :] = jnp.zeros((NPAD - n, 128), jnp.float32)

    p64 = lax.broadcasted_iota(jnp.int32, (n, 64), 1)
    i64 = (p64 * 37) >> 8
    j64 = p64 - i64 * 7
    pv = ((i64 < jnp.broadcast_to(ny, (n, 64)))
          & (j64 < jnp.broadcast_to(nx, (n, 64)))
          & (p64 < 49)).astype(jnp.int32)
    anyv = pv.max(axis=0, keepdims=True)
    pf_ref[...] = jnp.broadcast_to(anyv, (8, 64))


def _sc_kernel(mi_hbm, mf_hbm, gtl_hbm, gtr_hbm, gbl_hbm, gbr_hbm, oix_hbm,
               f0, f1, f2, f3, out_hbm,
               mi_v, mf_v, gib0, gib1, gib2, gib3, oib,
               tl_v, tr_v, bl_v, br_v, orow_v,
               semm, semg, semo):
    wid = lax.axis_index("s") * 2 + lax.axis_index("c")
    base = wid * RPW
    frefs = (f0, f1, f2, f3)
    gibs = (gib0, gib1, gib2, gib3)
    cbufs = (tl_v, tr_v, bl_v, br_v)
    lane = lax.iota(jnp.int32, 16)

    mcps = [pltpu.async_copy(src.at[pl.ds(base, RPW)], dst, semm)
            for src, dst in ((mi_hbm, mi_v), (mf_hbm, mf_v),
                             (gtl_hbm, gib0), (gtr_hbm, gib1),
                             (gbl_hbm, gib2), (gbr_hbm, gib3),
                             (oix_hbm, oib))]
    for cp in mcps:
        cp.wait()

    @pl.loop(0, RPW)
    def _(r):
        cnt = jnp.max(mi_v[r, pl.ds(80, 16)])

        @pl.when(cnt > 0)
        def _():
            lvl = jnp.max(mi_v[r, pl.ds(96, 16)])
            for L in range(4):
                @pl.when(lvl == L + 2)
                def _(L=L):
                    cps = [pltpu.async_copy(frefs[L].at[gibs[c].at[r]],
                                            cbufs[c], semg)
                           for c in range(4)]
                    for cp in cps:
                        cp.wait()
            wr = [mf_v[r, pl.ds(c * 16, 16)] for c in range(4)]

            @pl.loop(0, S)
            def _(s):
                spl = [jnp.ones((16,), jnp.float32)
                       * jnp.max(jnp.where(lane == s, wr[c], -1.0))
                       for c in range(4)]

                @pl.loop(0, 16)
                def _(ch):
                    off = ch * 16
                    val = (tl_v[s, pl.ds(off, 16)] * spl[0]
                           + tr_v[s, pl.ds(off, 16)] * spl[1]
                           + bl_v[s, pl.ds(off, 16)] * spl[2]
                           + br_v[s, pl.ds(off, 16)] * spl[3])
                    orow_v[s + S * (ch >> 3), pl.ds((ch & 7) * 16, 16)] = val

            pltpu.async_copy(orow_v, out_hbm.at[oib.at[r]], semo).wait()


def _finalize_kernel(pf_ref, o2d_hbm, mi_ref, out_ref, slo, shi, sem):
    n = out_ref.shape[2]
    i_p = pl.program_id(0)
    ny = mi_ref[0:n, 112:113]
    nx = mi_ref[0:n, 120:121]
    mi_row = i_p < ny
    zeros = jnp.zeros((n, 128), jnp.float32)
    for j in range(OUT):
        p = i_p * OUT + j

        @pl.when(pf_ref[p] > 0)
        def _(j=j, p=p):
            cl = pltpu.make_async_copy(
                o2d_hbm.at[pl.ds(p * n, n)], slo, sem)
            ch = pltpu.make_async_copy(
                o2d_hbm.at[pl.ds((49 + p) * n, n)], shi, sem)
            cl.start()
            ch.start()
            cl.wait()
            ch.wait()
            mask = mi_row & (j < nx)
            out_ref[0, j, :, 0:128] = jnp.where(mask, slo[...], 0.0)
            out_ref[0, j, :, 128:256] = jnp.where(mask, shi[...], 0.0)

        @pl.when(pf_ref[p] == 0)
        def _(j=j):
            out_ref[0, j, :, 0:128] = zeros
            out_ref[0, j, :, 128:256] = zeros


def kernel(feature_maps_0, feature_maps_1, feature_maps_2, feature_maps_3,
           rois):
    n = rois.shape[0]
    c = feature_maps_0.shape[-1]
    feats2d = [f.reshape(-1, c) for f in
               (feature_maps_0, feature_maps_1, feature_maps_2,
                feature_maps_3)]

    mi = pl.pallas_call(
        _meta_kernel,
        out_shape=[
            jax.ShapeDtypeStruct((NPAD, 128), jnp.int32),
            jax.ShapeDtypeStruct((NPAD, 128), jnp.float32),
        ] + [jax.ShapeDtypeStruct((NPAD, 16), jnp.int32)] * 4
          + [jax.ShapeDtypeStruct((NPAD, 32), jnp.int32),
             jax.ShapeDtypeStruct((8, 64), jnp.int32)],
    )(rois)

    cp = pltpu.CompilerParams()
    if "needs_layout_passes" in pltpu.CompilerParams.__dataclass_fields__:
        cp = dataclasses.replace(cp, needs_layout_passes=False)
    mesh = plsc.VectorSubcoreMesh(core_axis_name="c", subcore_axis_name="s")
    sc = pl.kernel(
        _sc_kernel,
        out_type=jax.ShapeDtypeStruct((2 * n * 49, c // 2), jnp.float32),
        mesh=mesh,
        compiler_params=cp,
        scratch_types=[
            pltpu.VMEM((RPW, 128), jnp.int32),
            pltpu.VMEM((RPW, 128), jnp.float32),
            pltpu.VMEM((RPW, 16), jnp.int32),
            pltpu.VMEM((RPW, 16), jnp.int32),
            pltpu.VMEM((RPW, 16), jnp.int32),
            pltpu.VMEM((RPW, 16), jnp.int32),
            pltpu.VMEM((RPW, 32), jnp.int32),
            pltpu.VMEM((S, 256), jnp.float32),
            pltpu.VMEM((S, 256), jnp.float32),
            pltpu.VMEM((S, 256), jnp.float32),
            pltpu.VMEM((S, 256), jnp.float32),
            pltpu.VMEM((2 * S, 128), jnp.float32),
            pltpu.SemaphoreType.DMA,
            pltpu.SemaphoreType.DMA,
            pltpu.SemaphoreType.DMA,
        ],
    )
    out2d = sc(*mi[:7], *feats2d)

    pflags = mi[7][0, :]
    outp = pl.pallas_call(
        _finalize_kernel,
        grid_spec=pltpu.PrefetchScalarGridSpec(
            num_scalar_prefetch=1,
            grid=(OUT,),
            in_specs=[
                pl.BlockSpec(memory_space=pl.ANY),
                pl.BlockSpec((NPAD, 128), lambda p, pf: (0, 0)),
            ],
            out_specs=pl.BlockSpec((1, OUT, n, c),
                                   lambda p, pf: (p, 0, 0, 0)),
            scratch_shapes=[
                pltpu.VMEM((n, 128), jnp.float32),
                pltpu.VMEM((n, 128), jnp.float32),
                pltpu.SemaphoreType.DMA,
            ],
        ),
        out_shape=jax.ShapeDtypeStruct((OUT, OUT, n, c), jnp.float32),
    )(pflags, out2d, mi[0])
    return jnp.transpose(outp, (2, 0, 1, 3))


# speedup vs baseline: 179.4546x; 1.0056x over previous
import dataclasses

import jax
import jax.numpy as jnp
from jax import lax
from jax.experimental import pallas as pl
from jax.experimental.pallas import tpu as pltpu
from jax.experimental.pallas import tpu_sc as plsc

OUT = 7
S = 16
NPAD = 1024
RPW = 32

_IDIV_M = [0] + [-(-256 // k) for k in range(1, 8)]


def _meta_kernel(rois_ref, mi_ref, mf_ref, gtl_ref, gtr_ref, gbl_ref,
                 gbr_ref, oix_ref, pf_ref):
    rois = rois_ref[...]
    n = rois.shape[0]
    x1 = rois[:, 1:2]
    y1 = rois[:, 2:3]
    x2 = rois[:, 3:4]
    y2 = rois[:, 4:5]
    roi_h = y2 - y1
    roi_w = x2 - x1
    lvl_f = jnp.log(jnp.sqrt(roi_h * roi_w) / 224.0) / jnp.log(2.0)
    level = jnp.clip(jnp.round(4.0 + lvl_f).astype(jnp.int32), 2, 5)
    box_b = rois[:, 0:1].astype(jnp.int32)

    i7 = lax.broadcasted_iota(jnp.int32, (1, OUT), 1).astype(jnp.float32)

    sinv = jnp.where(level == 2, 0.25,
           jnp.where(level == 3, 0.125,
           jnp.where(level == 4, 0.0625, 0.03125)))
    hm1f = jnp.where(level == 2, 255.0,
           jnp.where(level == 3, 127.0,
           jnp.where(level == 4, 63.0, 31.0)))
    hm1i = hm1f.astype(jnp.int32)
    hi = hm1i + 1
    y1c = x1 * sinv
    x1c = y1 * sinv
    y2c = x2 * sinv
    x2c = y2 * sinv
    hs = (y2c - y1c) * hm1f / (OUT - 1)
    ws = (x2c - x1c) * hm1f / (OUT - 1)
    in_y7 = y1c * hm1f + i7 * hs
    in_x7 = x1c * hm1f + i7 * ws
    vy = (in_y7 >= 0) & (in_y7 <= hm1f)
    vx = (in_x7 >= 0) & (in_x7 <= hm1f)
    ny = vy.astype(jnp.int32).sum(axis=1, keepdims=True)
    nx = vx.astype(jnp.int32).sum(axis=1, keepdims=True)
    cnt = ny * nx

    t16i = lax.broadcasted_iota(jnp.int32, (1, S), 1)
    nxm = jnp.maximum(nx, 1)
    i16 = jnp.zeros((n, S), jnp.int32)
    for k in range(1, 8):
        i16 = jnp.where(nxm == k, (t16i * _IDIV_M[k]) >> 8, i16)
    j16 = t16i - i16 * nxm
    valid = t16i < cnt

    i16f = i16.astype(jnp.float32)
    j16f = j16.astype(jnp.float32)
    in_y = y1c * hm1f + i16f * hs
    in_x = x1c * hm1f + j16f * ws
    top = jnp.floor(in_y)
    bot = jnp.ceil(in_y)
    lef = jnp.floor(in_x)
    rig = jnp.ceil(in_x)
    yl = in_y - top
    xl = in_x - lef
    ti = jnp.clip(top, 0, hm1f).astype(jnp.int32)
    bi = jnp.clip(bot, 0, hm1f).astype(jnp.int32)
    li = jnp.clip(lef, 0, hm1f).astype(jnp.int32)
    ri = jnp.clip(rig, 0, hm1f).astype(jnp.int32)
    rowt = (box_b * hi + ti) * hi
    rowb = (box_b * hi + bi) * hi
    idx_tl = rowt + li
    idx_tr = rowt + ri
    idx_bl = rowb + li
    idx_br = rowb + ri
    vf = valid.astype(jnp.float32)
    wtl = (1.0 - xl) * (1.0 - yl) * vf
    wtr = xl * (1.0 - yl) * vf
    wbl = (1.0 - xl) * yl * vf
    wbr = xl * yl * vf
    rid = lax.broadcasted_iota(jnp.int32, (n, S), 0)
    oidx = jnp.where(valid, (i16 * 7 + j16) * n + rid, 48 * n + rid)
    n49 = 49 * n

    gtl_ref[0:n, :] = idx_tl
    gtr_ref[0:n, :] = idx_tr
    gbl_ref[0:n, :] = idx_bl
    gbr_ref[0:n, :] = idx_br
    oix_ref[0:n, 0:16] = oidx
    oix_ref[0:n, 16:32] = oidx + n49
    oix_ref[n:NPAD, :] = jnp.zeros((NPAD - n, 32), jnp.int32)
    for ref in (gtl_ref, gtr_ref, gbl_ref, gbr_ref):
        ref[n:NPAD, :] = jnp.zeros((NPAD - n, 16), jnp.int32)
    mi_ref[0:n, 0:16] = idx_tl
    mi_ref[0:n, 16:32] = idx_tr
    mi_ref[0:n, 32:48] = idx_bl
    mi_ref[0:n, 48:64] = idx_br
    mi_ref[0:n, 64:80] = oidx
    mi_ref[0:n, 80:96] = jnp.broadcast_to(cnt, (n, 16))
    mi_ref[0:n, 96:112] = jnp.broadcast_to(level, (n, 16))
    mi_ref[0:n, 112:120] = jnp.broadcast_to(ny, (n, 8))
    mi_ref[0:n, 120:128] = jnp.broadcast_to(nx, (n, 8))
    mi_ref[n:NPAD, :] = jnp.zeros((NPAD - n, 128), jnp.int32)
    mf_ref[0:n, 0:16] = wtl
    mf_ref[0:n, 16:32] = wtr
    mf_ref[0:n, 32:48] = wbl
    mf_ref[0:n, 48:64] = wbr
    mf_ref[0:n, 64:128] = jnp.zeros((n, 64), jnp.float32)
    mf_ref[n:NPAD, :] = jnp.zeros((NPAD - n, 128), jnp.float32)

    p64 = lax.broadcasted_iota(jnp.int32, (n, 64), 1)
    i64 = (p64 * 37) >> 8
    j64 = p64 - i64 * 7
    pv = ((i64 < jnp.broadcast_to(ny, (n, 64)))
          & (j64 < jnp.broadcast_to(nx, (n, 64)))
          & (p64 < 49)).astype(jnp.int32)
    anyv = pv.max(axis=0, keepdims=True)
    pf_ref[...] = jnp.broadcast_to(anyv, (8, 64))


def _sc_kernel(mi_hbm, mf_hbm, gtl_hbm, gtr_hbm, gbl_hbm, gbr_hbm, oix_hbm,
               f0, f1, f2, f3, out_hbm,
               mi_v, mf_v, gib0, gib1, gib2, gib3, oib,
               tl_v, tr_v, bl_v, br_v, orow_v,
               semm, semg, semo):
    wid = lax.axis_index("s") * 2 + lax.axis_index("c")
    base = wid * RPW
    frefs = (f0, f1, f2, f3)
    gibs = (gib0, gib1, gib2, gib3)
    cbufs = (tl_v, tr_v, bl_v, br_v)
    lane = lax.iota(jnp.int32, 16)

    mcps = [pltpu.async_copy(src.at[pl.ds(base, RPW)], dst, semm)
            for src, dst in ((mi_hbm, mi_v), (mf_hbm, mf_v),
                             (gtl_hbm, gib0), (gtr_hbm, gib1),
                             (gbl_hbm, gib2), (gbr_hbm, gib3),
                             (oix_hbm, oib))]
    for cp in mcps:
        cp.wait()

    @pl.loop(0, RPW)
    def _(r):
        cnt = jnp.max(mi_v[r, pl.ds(80, 16)])

        @pl.when(cnt > 0)
        def _():
            lvl = jnp.max(mi_v[r, pl.ds(96, 16)])
            for L in range(4):
                @pl.when(lvl == L + 2)
                def _(L=L):
                    cps = [pltpu.async_copy(frefs[L].at[gibs[c].at[r]],
                                            cbufs[c], semg)
                           for c in range(4)]
                    for cp in cps:
                        cp.wait()
            wr = [mf_v[r, pl.ds(c * 16, 16)] for c in range(4)]

            @pl.loop(0, S)
            def _(s):
                spl = [jnp.ones((16,), jnp.float32)
                       * jnp.max(jnp.where(lane == s, wr[c], -1.0))
                       for c in range(4)]

                @pl.loop(0, 16)
                def _(ch):
                    off = ch * 16
                    val = (tl_v[s, pl.ds(off, 16)] * spl[0]
                           + tr_v[s, pl.ds(off, 16)] * spl[1]
                           + bl_v[s, pl.ds(off, 16)] * spl[2]
                           + br_v[s, pl.ds(off, 16)] * spl[3])
                    orow_v[s + S * (ch >> 3), pl.ds((ch & 7) * 16, 16)] = val

            pltpu.async_copy(orow_v, out_hbm.at[oib.at[r]], semo).wait()


def _finalize_kernel(pf_ref, o2d_hbm, mi_ref, out_ref, slo, shi, sem):
    n = out_ref.shape[2]
    i_p = pl.program_id(0)
    ny = mi_ref[0:n, 112:113]
    nx = mi_ref[0:n, 120:121]
    mi_row = i_p < ny
    zeros = jnp.zeros((n, 128), jnp.float32)
    for j in range(OUT):
        p = i_p * OUT + j

        @pl.when(pf_ref[p] > 0)
        def _(j=j, p=p):
            cl = pltpu.make_async_copy(
                o2d_hbm.at[pl.ds(p * n, n)], slo, sem)
            ch = pltpu.make_async_copy(
                o2d_hbm.at[pl.ds((49 + p) * n, n)], shi, sem)
            cl.start()
            ch.start()
            cl.wait()
            ch.wait()
            mask = mi_row & (j < nx)
            out_ref[0, j, :, 0:128] = jnp.where(mask, slo[...], 0.0)
            out_ref[0, j, :, 128:256] = jnp.where(mask, shi[...], 0.0)

        @pl.when(pf_ref[p] == 0)
        def _(j=j):
            out_ref[0, j, :, 0:128] = zeros
            out_ref[0, j, :, 128:256] = zeros


def kernel(feature_maps_0, feature_maps_1, feature_maps_2, feature_maps_3,
           rois):
    n = rois.shape[0]
    c = feature_maps_0.shape[-1]
    feats2d = [f.reshape(-1, c) for f in
               (feature_maps_0, feature_maps_1, feature_maps_2,
                feature_maps_3)]

    mi = pl.pallas_call(
        _meta_kernel,
        out_shape=[
            jax.ShapeDtypeStruct((NPAD, 128), jnp.int32),
            jax.ShapeDtypeStruct((NPAD, 128), jnp.float32),
        ] + [jax.ShapeDtypeStruct((NPAD, 16), jnp.int32)] * 4
          + [jax.ShapeDtypeStruct((NPAD, 32), jnp.int32),
             jax.ShapeDtypeStruct((8, 64), jnp.int32)],
    )(rois)

    cp = pltpu.CompilerParams()
    if "needs_layout_passes" in pltpu.CompilerParams.__dataclass_fields__:
        cp = dataclasses.replace(cp, needs_layout_passes=False)
    mesh = plsc.VectorSubcoreMesh(core_axis_name="c", subcore_axis_name="s")
    sc = pl.kernel(
        _sc_kernel,
        out_type=jax.ShapeDtypeStruct((2 * n * 49, c // 2), jnp.float32),
        mesh=mesh,
        compiler_params=cp,
        scratch_types=[
            pltpu.VMEM((RPW, 128), jnp.int32),
            pltpu.VMEM((RPW, 128), jnp.float32),
            pltpu.VMEM((RPW, 16), jnp.int32),
            pltpu.VMEM((RPW, 16), jnp.int32),
            pltpu.VMEM((RPW, 16), jnp.int32),
            pltpu.VMEM((RPW, 16), jnp.int32),
            pltpu.VMEM((RPW, 32), jnp.int32),
            pltpu.VMEM((S, 256), jnp.float32),
            pltpu.VMEM((S, 256), jnp.float32),
            pltpu.VMEM((S, 256), jnp.float32),
            pltpu.VMEM((S, 256), jnp.float32),
            pltpu.VMEM((2 * S, 128), jnp.float32),
            pltpu.SemaphoreType.DMA,
            pltpu.SemaphoreType.DMA,
            pltpu.SemaphoreType.DMA,
        ],
    )
    out2d = sc(*mi[:7], *feats2d)

    pflags = mi[7][0, :]
    outp = pl.pallas_call(
        _finalize_kernel,
        grid_spec=pltpu.PrefetchScalarGridSpec(
            num_scalar_prefetch=1,
            grid=(OUT,),
            in_specs=[
                pl.BlockSpec(memory_space=pl.ANY),
                pl.BlockSpec((NPAD, 128), lambda p, pf: (0, 0)),
            ],
            out_specs=pl.BlockSpec((1, OUT, n, c),
                                   lambda p, pf: (p, 0, 0, 0)),
            scratch_shapes=[
                pltpu.VMEM((n, 128), jnp.float32),
                pltpu.VMEM((n, 128), jnp.float32),
                pltpu.SemaphoreType.DMA,
            ],
        ),
        out_shape=jax.ShapeDtypeStruct((OUT, OUT, n, c), jnp.float32),
    )(pflags, out2d, mi[0])
    return jnp.transpose(outp, (2, 0, 1, 3))
